# Initial kernel scaffold; baseline (speedup 1.0000x reference)
import jax
import jax.numpy as jnp
from jax.experimental import pallas as pl


def kernel(features, edges, W1, b1, W2, b2):
    raise NotImplementedError("write your pallas kernel here")


# speedup vs baseline: 14.5326x; 14.5326x over previous
import functools

import jax
import jax.numpy as jnp
from jax import lax
from jax.experimental import pallas as pl
from jax.experimental.pallas import tpu as pltpu
from jax.experimental.pallas import tpu_sc as plsc

N = 10000
E = 320000
DF = 128
D1 = 64
D2 = 32
NPAD = 10240

NC = 2
NS = 16
NW = NC * NS
EPW = E // NW
K = 80
NCHUNK = EPW // K
RPT = NPAD // NS

_MESH = plsc.VectorSubcoreMesh(
    core_axis_name="c", subcore_axis_name="s", num_cores=NC, num_subcores=NS)
_SC_PARAMS = pltpu.CompilerParams(use_tc_tiling_on_sc=False)


DW = 16


def _make_deg_kernel():
  @functools.partial(
      pl.kernel,
      out_type=jax.ShapeDtypeStruct((NC * NPAD, DW), jnp.float32),
      mesh=_MESH,
      compiler_params=_SC_PARAMS,
      scratch_types=[
          pltpu.VMEM_SHARED((NPAD, DW), jnp.float32),
          pltpu.VMEM((K,), jnp.int32),
          pltpu.VMEM((K, DW), jnp.float32),
      ],
  )
  def deg_kernel(dst_hbm, zeros_hbm, ones_hbm, out_hbm, acc, didx, ones_v):
    c = lax.axis_index("c")
    s = lax.axis_index("s")
    r0 = s * RPT
    pltpu.sync_copy(zeros_hbm.at[pl.ds(r0, RPT), :], acc.at[pl.ds(r0, RPT), :])
    pltpu.sync_copy(ones_hbm, ones_v)
    plsc.subcore_barrier()
    base = (s * NC + c) * EPW

    def chunk(j, carry):
      off = base + j * K
      pltpu.sync_copy(dst_hbm.at[pl.ds(off, K)], didx)
      pltpu.sync_copy(ones_v, acc.at[didx], add=True)
      return carry

    lax.fori_loop(0, NCHUNK, chunk, 0)
    plsc.subcore_barrier()
    pltpu.sync_copy(acc.at[pl.ds(r0, RPT), :],
                    out_hbm.at[pl.ds(c * NPAD + r0, RPT), :])

  return deg_kernel


def _make_agg_kernel(d):
  @functools.partial(
      pl.kernel,
      out_type=jax.ShapeDtypeStruct((NC * NPAD, d), jnp.float32),
      mesh=_MESH,
      compiler_params=_SC_PARAMS,
      scratch_types=[
          pltpu.VMEM_SHARED((NPAD, d), jnp.float32),
          pltpu.VMEM((K,), jnp.int32),
          pltpu.VMEM((K,), jnp.int32),
          pltpu.VMEM((K, d), jnp.float32),
          pltpu.SemaphoreType.DMA,
      ],
  )
  def agg_kernel(tab_hbm, src_hbm, dst_hbm, zeros_hbm, out_hbm,
                 acc, sidx, didx, rows, sem):
    c = lax.axis_index("c")
    s = lax.axis_index("s")
    r0 = s * RPT
    pltpu.sync_copy(zeros_hbm.at[pl.ds(r0, RPT), :], acc.at[pl.ds(r0, RPT), :])
    plsc.subcore_barrier()
    base = (s * NC + c) * EPW

    def chunk(j, carry):
      off = base + j * K
      pltpu.sync_copy(src_hbm.at[pl.ds(off, K)], sidx)
      pltpu.sync_copy(dst_hbm.at[pl.ds(off, K)], didx)
      pltpu.async_copy(tab_hbm.at[sidx], rows, sem).wait()
      pltpu.sync_copy(rows, acc.at[didx], add=True)
      return carry

    lax.fori_loop(0, NCHUNK, chunk, 0)
    plsc.subcore_barrier()
    pltpu.sync_copy(acc.at[pl.ds(r0, RPT), :],
                    out_hbm.at[pl.ds(c * NPAD + r0, RPT), :])

  return agg_kernel


_deg_kernel = _make_deg_kernel()
_agg64 = _make_agg_kernel(D1)
_agg32 = _make_agg_kernel(D2)


BN = 1000


def _tc1_body(d0, d1, x, w1, hp_o, dis_o):
  deg = d0[...] + d1[...] + 1.0
  dis = lax.rsqrt(deg)
  h = jnp.dot(x[...], w1[...], preferred_element_type=jnp.float32)
  hp_o[...] = h * dis
  dis_o[...] = dis


_tc1 = pl.pallas_call(
    _tc1_body,
    grid=(N // BN,),
    in_specs=[
        pl.BlockSpec((BN, 1), lambda i: (i, 0)),
        pl.BlockSpec((BN, 1), lambda i: (i, 0)),
        pl.BlockSpec((BN, DF), lambda i: (i, 0)),
        pl.BlockSpec((DF, D1), lambda i: (0, 0)),
    ],
    out_specs=[
        pl.BlockSpec((BN, D1), lambda i: (i, 0)),
        pl.BlockSpec((BN, 1), lambda i: (i, 0)),
    ],
    out_shape=[
        jax.ShapeDtypeStruct((N, D1), jnp.float32),
        jax.ShapeDtypeStruct((N, 1), jnp.float32),
    ],
)


def _tc2_body(p0, p1, hp, dis, b1, w2, g_o):
  h1 = jnp.maximum(dis[...] * (p0[...] + p1[...] + hp[...]) + b1[...], 0.0)
  g_o[...] = jnp.dot(h1, w2[...], preferred_element_type=jnp.float32) * dis[...]


_tc2 = pl.pallas_call(
    _tc2_body,
    grid=(N // BN,),
    in_specs=[
        pl.BlockSpec((BN, D1), lambda i: (i, 0)),
        pl.BlockSpec((BN, D1), lambda i: (i, 0)),
        pl.BlockSpec((BN, D1), lambda i: (i, 0)),
        pl.BlockSpec((BN, 1), lambda i: (i, 0)),
        pl.BlockSpec((1, D1), lambda i: (0, 0)),
        pl.BlockSpec((D1, D2), lambda i: (0, 0)),
    ],
    out_specs=pl.BlockSpec((BN, D2), lambda i: (i, 0)),
    out_shape=jax.ShapeDtypeStruct((N, D2), jnp.float32),
)


def _tc3_body(q0, q1, g, dis, b2, o):
  h2 = dis[...] * (q0[...] + q1[...] + g[...]) + b2[...]
  m = jnp.max(h2, axis=1, keepdims=True)
  e = jnp.exp(h2 - m)
  lse = jnp.log(jnp.sum(e, axis=1, keepdims=True)) + m
  o[...] = h2 - lse


_tc3 = pl.pallas_call(
    _tc3_body,
    grid=(N // BN,),
    in_specs=[
        pl.BlockSpec((BN, D2), lambda i: (i, 0)),
        pl.BlockSpec((BN, D2), lambda i: (i, 0)),
        pl.BlockSpec((BN, D2), lambda i: (i, 0)),
        pl.BlockSpec((BN, 1), lambda i: (i, 0)),
        pl.BlockSpec((1, D2), lambda i: (0, 0)),
    ],
    out_specs=pl.BlockSpec((BN, D2), lambda i: (i, 0)),
    out_shape=jax.ShapeDtypeStruct((N, D2), jnp.float32),
)


def kernel(features, edges, W1, b1, W2, b2):
  src = edges[0].astype(jnp.int32)
  dst = edges[1].astype(jnp.int32)

  zerosd = jnp.zeros((NPAD, DW), jnp.float32)
  ones_k = jnp.ones((K, DW), jnp.float32)
  degp = _deg_kernel(dst, zerosd, ones_k)
  deg0 = degp[:N, :1]
  deg1 = degp[NPAD:NPAD + N, :1]

  hp, dis = _tc1(deg0, deg1, features, W1)

  zeros64 = jnp.zeros((NPAD, D1), jnp.float32)
  p = _agg64(hp, src, dst, zeros64)
  g = _tc2(p[:N], p[NPAD:NPAD + N], hp, dis, b1.reshape(1, D1), W2)

  zeros32 = jnp.zeros((NPAD, D2), jnp.float32)
  q = _agg32(g, src, dst, zeros32)
  out = _tc3(q[:N], q[NPAD:NPAD + N], g, dis, b2.reshape(1, D2))
  return out


# speedup vs baseline: 41.2835x; 2.8407x over previous
import functools

import jax
import jax.numpy as jnp
from jax import lax
from jax.experimental import pallas as pl
from jax.experimental.pallas import tpu as pltpu
from jax.experimental.pallas import tpu_sc as plsc

N = 10000
E = 320000
DF = 128
D1 = 64
D2 = 32
NPAD = 10240

NC = 2
NS = 16
NW = NC * NS
EPW = E // NW
K = 80
NCHUNK = EPW // K
RPT = NPAD // NS

_MESH = plsc.VectorSubcoreMesh(
    core_axis_name="c", subcore_axis_name="s", num_cores=NC, num_subcores=NS)
_SC_PARAMS = pltpu.CompilerParams(use_tc_tiling_on_sc=False)


DW = 16
NBUF = 5
NOUTER = NCHUNK // NBUF
DEGW = 6


def _make_deg_kernel():
  @functools.partial(
      pl.kernel,
      out_type=jax.ShapeDtypeStruct((NC * NPAD, DW), jnp.float32),
      mesh=_MESH,
      compiler_params=_SC_PARAMS,
      scratch_types=[
          pltpu.VMEM_SHARED((NPAD, DW), jnp.float32),
          pltpu.VMEM((NCHUNK, K), jnp.int32),
          pltpu.VMEM((K, DW), jnp.float32),
          pltpu.SemaphoreType.DMA,
      ],
  )
  def deg_kernel(dst4_hbm, zeros_hbm, ones_hbm, out_hbm, acc, didx3, ones_v,
                 dsem):
    c = lax.axis_index("c")
    s = lax.axis_index("s")
    r0 = s * RPT
    w = s * NC + c
    pltpu.sync_copy(zeros_hbm.at[pl.ds(r0, RPT), :], acc.at[pl.ds(r0, RPT), :])
    pltpu.sync_copy(dst4_hbm.at[w], didx3)
    pltpu.sync_copy(ones_hbm, ones_v)
    plsc.subcore_barrier()

    def drain_one():
      pltpu.make_async_copy(ones_v, acc.at[didx3.at[0]], dsem).wait()

    def loop(j, carry):
      pltpu.async_copy(ones_v, acc.at[didx3.at[j]], dsem, add=True)

      @pl.when(j >= DEGW)
      def _():
        drain_one()

      return carry

    lax.fori_loop(0, NCHUNK, loop, 0)

    def tail(i, carry):
      drain_one()
      return carry

    lax.fori_loop(0, DEGW, tail, 0)
    plsc.subcore_barrier()
    pltpu.sync_copy(acc.at[pl.ds(r0, RPT), :],
                    out_hbm.at[pl.ds(c * NPAD + r0, RPT), :])

  return deg_kernel


def _make_agg_kernel(d):
  scratch = [
      pltpu.VMEM_SHARED((NPAD, d), jnp.float32),
      pltpu.VMEM((EPW,), jnp.int32),
      pltpu.VMEM((NCHUNK, K), jnp.int32),
  ]
  scratch += [pltpu.VMEM((K, d), jnp.float32) for _ in range(NBUF)]
  scratch += [pltpu.SemaphoreType.DMA for _ in range(2 * NBUF)]

  @functools.partial(
      pl.kernel,
      out_type=jax.ShapeDtypeStruct((NC * NPAD, d), jnp.float32),
      mesh=_MESH,
      compiler_params=_SC_PARAMS,
      scratch_types=scratch,
  )
  def agg_kernel(tab_hbm, src_hbm, dst4_hbm, zeros_hbm, out_hbm,
                 acc, sidx, didx3, *bufs):
    rows = bufs[:NBUF]
    gsem = bufs[NBUF:2 * NBUF]
    ssem = bufs[2 * NBUF:3 * NBUF]
    c = lax.axis_index("c")
    s = lax.axis_index("s")
    r0 = s * RPT
    w = s * NC + c
    base = w * EPW
    pltpu.sync_copy(zeros_hbm.at[pl.ds(r0, RPT), :], acc.at[pl.ds(r0, RPT), :])
    pltpu.sync_copy(src_hbm.at[pl.ds(base, EPW)], sidx)
    pltpu.sync_copy(dst4_hbm.at[w], didx3)
    plsc.subcore_barrier()

    def gfire(j, b):
      pltpu.async_copy(tab_hbm.at[sidx.at[pl.ds(j * K, K)]], rows[b], gsem[b])

    def gwait(b):
      pltpu.make_async_copy(
          tab_hbm.at[sidx.at[pl.ds(0, K)]], rows[b], gsem[b]).wait()

    def sfire(j, b):
      pltpu.async_copy(rows[b], acc.at[didx3.at[j]], ssem[b], add=True)

    def swait(b):
      pltpu.make_async_copy(rows[b], acc.at[didx3.at[0]], ssem[b]).wait()

    for b in range(NBUF - 1):
      gfire(b, b)

    def outer(t, carry):
      for b in range(NBUF):
        j = t * NBUF + b
        b1 = (b - 1) % NBUF
        gwait(b)
        sfire(j, b)

        @pl.when(j > 0)
        def _():
          swait(b1)

        @pl.when(j + NBUF - 1 < NCHUNK)
        def _():
          gfire(j + NBUF - 1, b1)

      return carry

    lax.fori_loop(0, NOUTER, outer, 0)
    swait((NCHUNK - 1) % NBUF)
    plsc.subcore_barrier()
    pltpu.sync_copy(acc.at[pl.ds(r0, RPT), :],
                    out_hbm.at[pl.ds(c * NPAD + r0, RPT), :])

  return agg_kernel


_deg_kernel = _make_deg_kernel()
_agg64 = _make_agg_kernel(D1)
_agg32 = _make_agg_kernel(D2)


BN = 1000


def _tc1_body(d0, d1, x, w1, hp_o, dis_o):
  deg = d0[...] + d1[...] + 1.0
  dis = lax.rsqrt(deg)
  h = jnp.dot(x[...], w1[...], preferred_element_type=jnp.float32)
  hp_o[...] = h * dis
  dis_o[...] = dis


_tc1 = pl.pallas_call(
    _tc1_body,
    grid=(N // BN,),
    in_specs=[
        pl.BlockSpec((BN, 1), lambda i: (i, 0)),
        pl.BlockSpec((BN, 1), lambda i: (i, 0)),
        pl.BlockSpec((BN, DF), lambda i: (i, 0)),
        pl.BlockSpec((DF, D1), lambda i: (0, 0)),
    ],
    out_specs=[
        pl.BlockSpec((BN, D1), lambda i: (i, 0)),
        pl.BlockSpec((BN, 1), lambda i: (i, 0)),
    ],
    out_shape=[
        jax.ShapeDtypeStruct((N, D1), jnp.float32),
        jax.ShapeDtypeStruct((N, 1), jnp.float32),
    ],
)


def _tc2_body(p0, p1, hp, dis, b1, w2, g_o):
  h1 = jnp.maximum(dis[...] * (p0[...] + p1[...] + hp[...]) + b1[...], 0.0)
  g_o[...] = jnp.dot(h1, w2[...], preferred_element_type=jnp.float32) * dis[...]


_tc2 = pl.pallas_call(
    _tc2_body,
    grid=(N // BN,),
    in_specs=[
        pl.BlockSpec((BN, D1), lambda i: (i, 0)),
        pl.BlockSpec((BN, D1), lambda i: (i, 0)),
        pl.BlockSpec((BN, D1), lambda i: (i, 0)),
        pl.BlockSpec((BN, 1), lambda i: (i, 0)),
        pl.BlockSpec((1, D1), lambda i: (0, 0)),
        pl.BlockSpec((D1, D2), lambda i: (0, 0)),
    ],
    out_specs=pl.BlockSpec((BN, D2), lambda i: (i, 0)),
    out_shape=jax.ShapeDtypeStruct((N, D2), jnp.float32),
)


def _tc3_body(q0, q1, g, dis, b2, o):
  h2 = dis[...] * (q0[...] + q1[...] + g[...]) + b2[...]
  m = jnp.max(h2, axis=1, keepdims=True)
  e = jnp.exp(h2 - m)
  lse = jnp.log(jnp.sum(e, axis=1, keepdims=True)) + m
  o[...] = h2 - lse


_tc3 = pl.pallas_call(
    _tc3_body,
    grid=(N // BN,),
    in_specs=[
        pl.BlockSpec((BN, D2), lambda i: (i, 0)),
        pl.BlockSpec((BN, D2), lambda i: (i, 0)),
        pl.BlockSpec((BN, D2), lambda i: (i, 0)),
        pl.BlockSpec((BN, 1), lambda i: (i, 0)),
        pl.BlockSpec((1, D2), lambda i: (0, 0)),
    ],
    out_specs=pl.BlockSpec((BN, D2), lambda i: (i, 0)),
    out_shape=jax.ShapeDtypeStruct((N, D2), jnp.float32),
)


def kernel(features, edges, W1, b1, W2, b2):
  src = edges[0].astype(jnp.int32)
  dst = edges[1].astype(jnp.int32)
  dst4 = dst.reshape(NW, NCHUNK, K)

  zerosd = jnp.zeros((NPAD, DW), jnp.float32)
  ones_k = jnp.ones((K, DW), jnp.float32)
  degp = _deg_kernel(dst4, zerosd, ones_k)
  deg0 = degp[:N, :1]
  deg1 = degp[NPAD:NPAD + N, :1]

  hp, dis = _tc1(deg0, deg1, features, W1)

  zeros64 = jnp.zeros((NPAD, D1), jnp.float32)
  p = _agg64(hp, src, dst4, zeros64)
  g = _tc2(p[:N], p[NPAD:NPAD + N], hp, dis, b1.reshape(1, D1), W2)

  zeros32 = jnp.zeros((NPAD, D2), jnp.float32)
  q = _agg32(g, src, dst4, zeros32)
  out = _tc3(q[:N], q[NPAD:NPAD + N], g, dis, b2.reshape(1, D2))
  return out


# speedup vs baseline: 44.2128x; 1.0710x over previous
import functools

import jax
import jax.numpy as jnp
from jax import lax
from jax.experimental import pallas as pl
from jax.experimental.pallas import tpu as pltpu
from jax.experimental.pallas import tpu_sc as plsc

N = 10000
E = 320000
DF = 128
D1 = 64
D2 = 32
NPAD = 10000

NC = 2
NS = 16
NW = NC * NS
EPW = E // NW
K = 80
NCHUNK = EPW // K
RPT = NPAD // NS

_MESH = plsc.VectorSubcoreMesh(
    core_axis_name="c", subcore_axis_name="s", num_cores=NC, num_subcores=NS)
_SC_PARAMS = pltpu.CompilerParams(use_tc_tiling_on_sc=False)


DW = 16
NBUF = 5
NOUTER = NCHUNK // NBUF
DEGW = 6


def _make_deg_kernel():
  @functools.partial(
      pl.kernel,
      out_type=jax.ShapeDtypeStruct((NC * NPAD, 8), jnp.float32),
      mesh=_MESH,
      compiler_params=_SC_PARAMS,
      scratch_types=[
          pltpu.VMEM_SHARED((NPAD, DW), jnp.float32),
          pltpu.VMEM((NCHUNK, K), jnp.int32),
          pltpu.VMEM((K, DW), jnp.float32),
          pltpu.SemaphoreType.DMA,
      ],
  )
  def deg_kernel(dst4_hbm, zeros_hbm, ones_hbm, out_hbm, acc, didx3, ones_v,
                 dsem):
    c = lax.axis_index("c")
    s = lax.axis_index("s")
    r0 = s * RPT
    w = s * NC + c
    pltpu.sync_copy(zeros_hbm.at[pl.ds(r0, RPT), :], acc.at[pl.ds(r0, RPT), :])
    pltpu.sync_copy(dst4_hbm.at[w], didx3)
    pltpu.sync_copy(ones_hbm, ones_v)
    plsc.subcore_barrier()

    def drain_one():
      pltpu.make_async_copy(ones_v, acc.at[didx3.at[0]], dsem).wait()

    def loop(j, carry):
      pltpu.async_copy(ones_v, acc.at[didx3.at[j]], dsem, add=True)

      @pl.when(j >= DEGW)
      def _():
        drain_one()

      return carry

    lax.fori_loop(0, NCHUNK, loop, 0)

    def tail(i, carry):
      drain_one()
      return carry

    lax.fori_loop(0, DEGW, tail, 0)
    plsc.subcore_barrier()
    pltpu.sync_copy(acc.at[pl.ds(r0, RPT), pl.ds(0, 8)],
                    out_hbm.at[pl.ds(c * NPAD + r0, RPT), :])

  return deg_kernel


def _make_agg_kernel(d):
  scratch = [
      pltpu.VMEM_SHARED((NPAD, d), jnp.float32),
      pltpu.VMEM((EPW,), jnp.int32),
      pltpu.VMEM((NCHUNK, K), jnp.int32),
  ]
  scratch += [pltpu.VMEM((K, d), jnp.float32) for _ in range(NBUF)]
  scratch += [pltpu.SemaphoreType.DMA for _ in range(2 * NBUF)]

  @functools.partial(
      pl.kernel,
      out_type=jax.ShapeDtypeStruct((NC * NPAD, d), jnp.float32),
      mesh=_MESH,
      compiler_params=_SC_PARAMS,
      scratch_types=scratch,
  )
  def agg_kernel(tab_hbm, src_hbm, dst4_hbm, zeros_hbm, out_hbm,
                 acc, sidx, didx3, *bufs):
    rows = bufs[:NBUF]
    gsem = bufs[NBUF:2 * NBUF]
    ssem = bufs[2 * NBUF:3 * NBUF]
    c = lax.axis_index("c")
    s = lax.axis_index("s")
    r0 = s * RPT
    w = s * NC + c
    base = w * EPW
    pltpu.sync_copy(zeros_hbm.at[pl.ds(r0, RPT), :], acc.at[pl.ds(r0, RPT), :])
    pltpu.sync_copy(src_hbm.at[pl.ds(base, EPW)], sidx)
    pltpu.sync_copy(dst4_hbm.at[w], didx3)
    plsc.subcore_barrier()

    def gfire(j, b):
      pltpu.async_copy(tab_hbm.at[sidx.at[pl.ds(j * K, K)]], rows[b], gsem[b])

    def gwait(b):
      pltpu.make_async_copy(
          tab_hbm.at[sidx.at[pl.ds(0, K)]], rows[b], gsem[b]).wait()

    def sfire(j, b):
      pltpu.async_copy(rows[b], acc.at[didx3.at[j]], ssem[b], add=True)

    def swait(b):
      pltpu.make_async_copy(rows[b], acc.at[didx3.at[0]], ssem[b]).wait()

    for b in range(NBUF - 1):
      gfire(b, b)

    def outer(t, carry):
      for b in range(NBUF):
        j = t * NBUF + b
        b1 = (b - 1) % NBUF
        gwait(b)
        sfire(j, b)

        @pl.when(j > 0)
        def _():
          swait(b1)

        @pl.when(j + NBUF - 1 < NCHUNK)
        def _():
          gfire(j + NBUF - 1, b1)

      return carry

    lax.fori_loop(0, NOUTER, outer, 0)
    swait((NCHUNK - 1) % NBUF)
    plsc.subcore_barrier()
    pltpu.sync_copy(acc.at[pl.ds(r0, RPT), :],
                    out_hbm.at[pl.ds(c * NPAD + r0, RPT), :])

  return agg_kernel


_deg_kernel = _make_deg_kernel()
_agg64 = _make_agg_kernel(D1)
_agg32 = _make_agg_kernel(D2)


BN = 1000


def _tc1_body(d0, d1, x, w1, hp_o, dis_o):
  deg = d0[...][:, :1] + d1[...][:, :1] + 1.0
  dis = lax.rsqrt(deg)
  h = jnp.dot(x[...], w1[...], preferred_element_type=jnp.float32)
  hp_o[...] = h * dis
  dis_o[...] = dis


_tc1 = pl.pallas_call(
    _tc1_body,
    grid=(N // BN,),
    in_specs=[
        pl.BlockSpec((BN, 8), lambda i: (i, 0)),
        pl.BlockSpec((BN, 8), lambda i: (N // BN + i, 0)),
        pl.BlockSpec((BN, DF), lambda i: (i, 0)),
        pl.BlockSpec((DF, D1), lambda i: (0, 0)),
    ],
    out_specs=[
        pl.BlockSpec((BN, D1), lambda i: (i, 0)),
        pl.BlockSpec((BN, 1), lambda i: (i, 0)),
    ],
    out_shape=[
        jax.ShapeDtypeStruct((N, D1), jnp.float32),
        jax.ShapeDtypeStruct((N, 1), jnp.float32),
    ],
)


def _tc2_body(p0, p1, hp, dis, b1, w2, g_o):
  h1 = jnp.maximum(dis[...] * (p0[...] + p1[...] + hp[...]) + b1[...], 0.0)
  g_o[...] = jnp.dot(h1, w2[...], preferred_element_type=jnp.float32) * dis[...]


_tc2 = pl.pallas_call(
    _tc2_body,
    grid=(N // BN,),
    in_specs=[
        pl.BlockSpec((BN, D1), lambda i: (i, 0)),
        pl.BlockSpec((BN, D1), lambda i: (N // BN + i, 0)),
        pl.BlockSpec((BN, D1), lambda i: (i, 0)),
        pl.BlockSpec((BN, 1), lambda i: (i, 0)),
        pl.BlockSpec((1, D1), lambda i: (0, 0)),
        pl.BlockSpec((D1, D2), lambda i: (0, 0)),
    ],
    out_specs=pl.BlockSpec((BN, D2), lambda i: (i, 0)),
    out_shape=jax.ShapeDtypeStruct((N, D2), jnp.float32),
)


def _tc3_body(q0, q1, g, dis, b2, o):
  h2 = dis[...] * (q0[...] + q1[...] + g[...]) + b2[...]
  m = jnp.max(h2, axis=1, keepdims=True)
  e = jnp.exp(h2 - m)
  lse = jnp.log(jnp.sum(e, axis=1, keepdims=True)) + m
  o[...] = h2 - lse


_tc3 = pl.pallas_call(
    _tc3_body,
    grid=(N // BN,),
    in_specs=[
        pl.BlockSpec((BN, D2), lambda i: (i, 0)),
        pl.BlockSpec((BN, D2), lambda i: (N // BN + i, 0)),
        pl.BlockSpec((BN, D2), lambda i: (i, 0)),
        pl.BlockSpec((BN, 1), lambda i: (i, 0)),
        pl.BlockSpec((1, D2), lambda i: (0, 0)),
    ],
    out_specs=pl.BlockSpec((BN, D2), lambda i: (i, 0)),
    out_shape=jax.ShapeDtypeStruct((N, D2), jnp.float32),
)


def kernel(features, edges, W1, b1, W2, b2):
  src = edges[0].astype(jnp.int32)
  dst = edges[1].astype(jnp.int32)
  dst4 = dst.reshape(NW, NCHUNK, K)

  zerosd = jnp.zeros((NPAD, DW), jnp.float32)
  ones_k = jnp.ones((K, DW), jnp.float32)
  degp = _deg_kernel(dst4, zerosd, ones_k)

  hp, dis = _tc1(degp, degp, features, W1)

  zeros64 = jnp.zeros((NPAD, D1), jnp.float32)
  p = _agg64(hp, src, dst4, zeros64)
  g = _tc2(p, p, hp, dis, b1.reshape(1, D1), W2)

  zeros32 = jnp.zeros((NPAD, D2), jnp.float32)
  q = _agg32(g, src, dst4, zeros32)
  out = _tc3(q, q, g, dis, b2.reshape(1, D2))
  return out


# speedup vs baseline: 47.9664x; 1.0849x over previous
import functools

import jax
import jax.numpy as jnp
from jax import lax
from jax.experimental import pallas as pl
from jax.experimental.pallas import tpu as pltpu
from jax.experimental.pallas import tpu_sc as plsc

N = 10000
E = 320000
DF = 128
D1 = 64
D2 = 32
NPAD = 10000

NC = 2
NS = 16
NW = NC * NS
K = 125
NCHUNK = 80
RPT = NPAD // NS

_MESH = plsc.VectorSubcoreMesh(
    core_axis_name="c", subcore_axis_name="s", num_cores=NC, num_subcores=NS)
_SC_PARAMS = pltpu.CompilerParams(use_tc_tiling_on_sc=False)


DW = 8
NBUF = 5
NOUTER = NCHUNK // NBUF
DEGW = 6


def _make_deg_kernel():
  @functools.partial(
      pl.kernel,
      out_type=jax.ShapeDtypeStruct((NC * NPAD, 8), jnp.float32),
      mesh=_MESH,
      compiler_params=_SC_PARAMS,
      scratch_types=[
          pltpu.VMEM_SHARED((NPAD, DW), jnp.float32),
          pltpu.VMEM((NCHUNK, K), jnp.int32),
          pltpu.VMEM((K, DW), jnp.float32),
          pltpu.SemaphoreType.DMA,
      ],
  )
  def deg_kernel(dst4_hbm, zeros_hbm, ones_hbm, out_hbm, acc, didx3, ones_v,
                 dsem):
    c = lax.axis_index("c")
    s = lax.axis_index("s")
    r0 = s * RPT
    w = s * NC + c
    pltpu.sync_copy(zeros_hbm.at[pl.ds(r0, RPT), :], acc.at[pl.ds(r0, RPT), :])
    pltpu.sync_copy(dst4_hbm.at[w], didx3)
    pltpu.sync_copy(ones_hbm, ones_v)
    plsc.subcore_barrier()

    def drain_one():
      pltpu.make_async_copy(ones_v, acc.at[didx3.at[0]], dsem).wait()

    def loop(j, carry):
      pltpu.async_copy(ones_v, acc.at[didx3.at[j]], dsem, add=True)

      @pl.when(j >= DEGW)
      def _():
        drain_one()

      return carry

    lax.fori_loop(0, NCHUNK, loop, 0)

    def tail(i, carry):
      drain_one()
      return carry

    lax.fori_loop(0, DEGW, tail, 0)
    plsc.subcore_barrier()
    pltpu.sync_copy(acc.at[pl.ds(r0, RPT), pl.ds(0, 8)],
                    out_hbm.at[pl.ds(c * NPAD + r0, RPT), :])

  return deg_kernel


def _make_agg_kernel(d):
  scratch = [
      pltpu.VMEM_SHARED((NPAD, d), jnp.float32),
      pltpu.VMEM((NCHUNK, K), jnp.int32),
      pltpu.VMEM((NCHUNK, K), jnp.int32),
  ]
  scratch += [pltpu.VMEM((K, d), jnp.float32) for _ in range(NBUF)]
  scratch += [pltpu.SemaphoreType.DMA for _ in range(2 * NBUF)]

  @functools.partial(
      pl.kernel,
      out_type=jax.ShapeDtypeStruct((NC * NPAD, d), jnp.float32),
      mesh=_MESH,
      compiler_params=_SC_PARAMS,
      scratch_types=scratch,
  )
  def agg_kernel(tab_hbm, src4_hbm, dst4_hbm, zeros_hbm, out_hbm,
                 acc, sidx3, didx3, *bufs):
    rows = bufs[:NBUF]
    gsem = bufs[NBUF:2 * NBUF]
    ssem = bufs[2 * NBUF:3 * NBUF]
    c = lax.axis_index("c")
    s = lax.axis_index("s")
    r0 = s * RPT
    w = s * NC + c
    pltpu.sync_copy(zeros_hbm.at[pl.ds(r0, RPT), :], acc.at[pl.ds(r0, RPT), :])
    pltpu.sync_copy(src4_hbm.at[w], sidx3)
    pltpu.sync_copy(dst4_hbm.at[w], didx3)
    plsc.subcore_barrier()

    def gfire(j, b):
      pltpu.async_copy(tab_hbm.at[sidx3.at[j]], rows[b], gsem[b])

    def gwait(b):
      pltpu.make_async_copy(
          tab_hbm.at[sidx3.at[0]], rows[b], gsem[b]).wait()

    def sfire(j, b):
      pltpu.async_copy(rows[b], acc.at[didx3.at[j]], ssem[b], add=True)

    def swait(b):
      pltpu.make_async_copy(rows[b], acc.at[didx3.at[0]], ssem[b]).wait()

    for b in range(NBUF - 1):
      gfire(b, b)

    def outer(t, carry):
      for b in range(NBUF):
        j = t * NBUF + b
        b1 = (b - 1) % NBUF
        gwait(b)
        sfire(j, b)

        @pl.when(j > 0)
        def _():
          swait(b1)

        @pl.when(j + NBUF - 1 < NCHUNK)
        def _():
          gfire(j + NBUF - 1, b1)

      return carry

    lax.fori_loop(0, NOUTER, outer, 0)
    swait((NCHUNK - 1) % NBUF)
    plsc.subcore_barrier()
    pltpu.sync_copy(acc.at[pl.ds(r0, RPT), :],
                    out_hbm.at[pl.ds(c * NPAD + r0, RPT), :])

  return agg_kernel


_deg_kernel = _make_deg_kernel()
_agg64 = _make_agg_kernel(D1)
_agg32 = _make_agg_kernel(D2)


BN = 2000


def _tc1_body(d0, d1, x, w1, hp_o, dis_o):
  deg = d0[...][:, :1] + d1[...][:, :1] + 1.0
  dis = lax.rsqrt(deg)
  h = jnp.dot(x[...], w1[...], preferred_element_type=jnp.float32)
  hp_o[...] = h * dis
  dis_o[...] = dis


_tc1 = pl.pallas_call(
    _tc1_body,
    grid=(N // BN,),
    in_specs=[
        pl.BlockSpec((BN, 8), lambda i: (i, 0)),
        pl.BlockSpec((BN, 8), lambda i: (NPAD // BN + i, 0)),
        pl.BlockSpec((BN, DF), lambda i: (i, 0)),
        pl.BlockSpec((DF, D1), lambda i: (0, 0)),
    ],
    out_specs=[
        pl.BlockSpec((BN, D1), lambda i: (i, 0)),
        pl.BlockSpec((BN, 1), lambda i: (i, 0)),
    ],
    out_shape=[
        jax.ShapeDtypeStruct((N, D1), jnp.float32),
        jax.ShapeDtypeStruct((N, 1), jnp.float32),
    ],
)


def _tc2_body(p0, p1, hp, dis, b1, w2, g_o):
  h1 = jnp.maximum(dis[...] * (p0[...] + p1[...] + hp[...]) + b1[...], 0.0)
  g_o[...] = jnp.dot(h1, w2[...], preferred_element_type=jnp.float32) * dis[...]


_tc2 = pl.pallas_call(
    _tc2_body,
    grid=(N // BN,),
    in_specs=[
        pl.BlockSpec((BN, D1), lambda i: (i, 0)),
        pl.BlockSpec((BN, D1), lambda i: (NPAD // BN + i, 0)),
        pl.BlockSpec((BN, D1), lambda i: (i, 0)),
        pl.BlockSpec((BN, 1), lambda i: (i, 0)),
        pl.BlockSpec((1, D1), lambda i: (0, 0)),
        pl.BlockSpec((D1, D2), lambda i: (0, 0)),
    ],
    out_specs=pl.BlockSpec((BN, D2), lambda i: (i, 0)),
    out_shape=jax.ShapeDtypeStruct((N, D2), jnp.float32),
)


def _tc3_body(q0, q1, g, dis, b2, o):
  h2 = dis[...] * (q0[...] + q1[...] + g[...]) + b2[...]
  m = jnp.max(h2, axis=1, keepdims=True)
  e = jnp.exp(h2 - m)
  lse = jnp.log(jnp.sum(e, axis=1, keepdims=True)) + m
  o[...] = h2 - lse


_tc3 = pl.pallas_call(
    _tc3_body,
    grid=(N // BN,),
    in_specs=[
        pl.BlockSpec((BN, D2), lambda i: (i, 0)),
        pl.BlockSpec((BN, D2), lambda i: (NPAD // BN + i, 0)),
        pl.BlockSpec((BN, D2), lambda i: (i, 0)),
        pl.BlockSpec((BN, 1), lambda i: (i, 0)),
        pl.BlockSpec((1, D2), lambda i: (0, 0)),
    ],
    out_specs=pl.BlockSpec((BN, D2), lambda i: (i, 0)),
    out_shape=jax.ShapeDtypeStruct((N, D2), jnp.float32),
)


def kernel(features, edges, W1, b1, W2, b2):
  src4 = edges[0].astype(jnp.int32).reshape(NW, NCHUNK, K)
  dst4 = edges[1].astype(jnp.int32).reshape(NW, NCHUNK, K)

  zerosd = jnp.zeros((NPAD, DW), jnp.float32)
  ones_k = jnp.ones((K, DW), jnp.float32)
  degp = _deg_kernel(dst4, zerosd, ones_k)

  hp, dis = _tc1(degp, degp, features, W1)

  zeros64 = jnp.zeros((NPAD, D1), jnp.float32)
  p = _agg64(hp, src4, dst4, zeros64)
  g = _tc2(p, p, hp, dis, b1.reshape(1, D1), W2)

  zeros32 = jnp.zeros((NPAD, D2), jnp.float32)
  q = _agg32(g, src4, dst4, zeros32)
  out = _tc3(q, q, g, dis, b2.reshape(1, D2))
  return out


# speedup vs baseline: 55.5666x; 1.1584x over previous
import functools

import jax
import jax.numpy as jnp
from jax import lax
from jax.experimental import pallas as pl
from jax.experimental.pallas import tpu as pltpu
from jax.experimental.pallas import tpu_sc as plsc

N = 10000
E = 320000
DF = 128
D1 = 64
D2 = 32
NPAD = 10000

NC = 2
NS = 16
NW = NC * NS
K = 125
NCHUNK = 80
RPT = NPAD // NS

_MESH = plsc.VectorSubcoreMesh(
    core_axis_name="c", subcore_axis_name="s", num_cores=NC, num_subcores=NS)
_SC_PARAMS = pltpu.CompilerParams(use_tc_tiling_on_sc=False)


DW = 8
NBUF = 5
NOUTER = NCHUNK // NBUF
DEGW = 6


def _make_deg_kernel():
  @functools.partial(
      pl.kernel,
      out_type=jax.ShapeDtypeStruct((NC * NPAD, 8), jnp.float32),
      mesh=_MESH,
      compiler_params=_SC_PARAMS,
      scratch_types=[
          pltpu.VMEM_SHARED((NPAD, DW), jnp.float32),
          pltpu.VMEM((NCHUNK, K), jnp.int32),
          pltpu.VMEM((K, DW), jnp.float32),
          pltpu.SemaphoreType.DMA,
      ],
  )
  def deg_kernel(dst4_hbm, zeros_hbm, ones_hbm, out_hbm, acc, didx3, ones_v,
                 dsem):
    c = lax.axis_index("c")
    s = lax.axis_index("s")
    r0 = s * RPT
    w = s * NC + c
    pltpu.sync_copy(zeros_hbm.at[pl.ds(r0, RPT), :], acc.at[pl.ds(r0, RPT), :])
    pltpu.sync_copy(dst4_hbm.at[w], didx3)
    pltpu.sync_copy(ones_hbm, ones_v)
    plsc.subcore_barrier()

    def drain_one():
      pltpu.make_async_copy(ones_v, acc.at[didx3.at[0]], dsem).wait()

    def loop(j, carry):
      pltpu.async_copy(ones_v, acc.at[didx3.at[j]], dsem, add=True)

      @pl.when(j >= DEGW)
      def _():
        drain_one()

      return carry

    lax.fori_loop(0, NCHUNK, loop, 0)

    def tail(i, carry):
      drain_one()
      return carry

    lax.fori_loop(0, DEGW, tail, 0)
    plsc.subcore_barrier()
    pltpu.sync_copy(acc.at[pl.ds(r0, RPT), pl.ds(0, 8)],
                    out_hbm.at[pl.ds(c * NPAD + r0, RPT), :])

  return deg_kernel


def _make_agg_kernel(d):
  scratch = [
      pltpu.VMEM_SHARED((NPAD, d), jnp.bfloat16),
      pltpu.VMEM((NCHUNK, K), jnp.int32),
      pltpu.VMEM((NCHUNK, K), jnp.int32),
  ]
  scratch += [pltpu.VMEM((K, d), jnp.bfloat16) for _ in range(NBUF)]
  scratch += [pltpu.SemaphoreType.DMA for _ in range(2 * NBUF)]

  @functools.partial(
      pl.kernel,
      out_type=jax.ShapeDtypeStruct((NC * NPAD, d), jnp.bfloat16),
      mesh=_MESH,
      compiler_params=_SC_PARAMS,
      scratch_types=scratch,
  )
  def agg_kernel(tab_hbm, src4_hbm, dst4_hbm, zeros_hbm, out_hbm,
                 acc, sidx3, didx3, *bufs):
    rows = bufs[:NBUF]
    gsem = bufs[NBUF:2 * NBUF]
    ssem = bufs[2 * NBUF:3 * NBUF]
    c = lax.axis_index("c")
    s = lax.axis_index("s")
    r0 = s * RPT
    w = s * NC + c
    pltpu.sync_copy(zeros_hbm.at[pl.ds(r0, RPT), :], acc.at[pl.ds(r0, RPT), :])
    pltpu.sync_copy(src4_hbm.at[w], sidx3)
    pltpu.sync_copy(dst4_hbm.at[w], didx3)
    plsc.subcore_barrier()

    def gfire(j, b):
      pltpu.async_copy(tab_hbm.at[sidx3.at[j]], rows[b], gsem[b])

    def gwait(b):
      pltpu.make_async_copy(
          tab_hbm.at[sidx3.at[0]], rows[b], gsem[b]).wait()

    def sfire(j, b):
      pltpu.async_copy(rows[b], acc.at[didx3.at[j]], ssem[b], add=True)

    def swait(b):
      pltpu.make_async_copy(rows[b], acc.at[didx3.at[0]], ssem[b]).wait()

    for b in range(NBUF - 1):
      gfire(b, b)

    def outer(t, carry):
      for b in range(NBUF):
        j = t * NBUF + b
        b1 = (b - 1) % NBUF
        gwait(b)
        sfire(j, b)

        @pl.when(j > 0)
        def _():
          swait(b1)

        @pl.when(j + NBUF - 1 < NCHUNK)
        def _():
          gfire(j + NBUF - 1, b1)

      return carry

    lax.fori_loop(0, NOUTER, outer, 0)
    swait((NCHUNK - 1) % NBUF)
    plsc.subcore_barrier()
    pltpu.sync_copy(acc.at[pl.ds(r0, RPT), :],
                    out_hbm.at[pl.ds(c * NPAD + r0, RPT), :])

  return agg_kernel


_deg_kernel = _make_deg_kernel()
_agg64 = _make_agg_kernel(D1)
_agg32 = _make_agg_kernel(D2)


BN = 2000


def _tc1_body(d0, d1, x, w1, hp_o, dis_o):
  deg = d0[...][:, :1] + d1[...][:, :1] + 1.0
  dis = lax.rsqrt(deg)
  h = jnp.dot(x[...], w1[...], preferred_element_type=jnp.float32)
  hp_o[...] = (h * dis).astype(jnp.bfloat16)
  dis_o[...] = dis


_tc1 = pl.pallas_call(
    _tc1_body,
    grid=(N // BN,),
    in_specs=[
        pl.BlockSpec((BN, 8), lambda i: (i, 0)),
        pl.BlockSpec((BN, 8), lambda i: (NPAD // BN + i, 0)),
        pl.BlockSpec((BN, DF), lambda i: (i, 0)),
        pl.BlockSpec((DF, D1), lambda i: (0, 0)),
    ],
    out_specs=[
        pl.BlockSpec((BN, D1), lambda i: (i, 0)),
        pl.BlockSpec((BN, 1), lambda i: (i, 0)),
    ],
    out_shape=[
        jax.ShapeDtypeStruct((N, D1), jnp.bfloat16),
        jax.ShapeDtypeStruct((N, 1), jnp.float32),
    ],
)


def _tc2_body(p0, p1, hp, dis, b1, w2, g_o):
  s = (p0[...].astype(jnp.float32) + p1[...].astype(jnp.float32)
       + hp[...].astype(jnp.float32))
  h1 = jnp.maximum(dis[...] * s + b1[...], 0.0)
  g = jnp.dot(h1, w2[...], preferred_element_type=jnp.float32) * dis[...]
  g_o[...] = g.astype(jnp.bfloat16)


_tc2 = pl.pallas_call(
    _tc2_body,
    grid=(N // BN,),
    in_specs=[
        pl.BlockSpec((BN, D1), lambda i: (i, 0)),
        pl.BlockSpec((BN, D1), lambda i: (NPAD // BN + i, 0)),
        pl.BlockSpec((BN, D1), lambda i: (i, 0)),
        pl.BlockSpec((BN, 1), lambda i: (i, 0)),
        pl.BlockSpec((1, D1), lambda i: (0, 0)),
        pl.BlockSpec((D1, D2), lambda i: (0, 0)),
    ],
    out_specs=pl.BlockSpec((BN, D2), lambda i: (i, 0)),
    out_shape=jax.ShapeDtypeStruct((N, D2), jnp.bfloat16),
)


def _tc3_body(q0, q1, g, dis, b2, o):
  s = (q0[...].astype(jnp.float32) + q1[...].astype(jnp.float32)
       + g[...].astype(jnp.float32))
  h2 = dis[...] * s + b2[...]
  m = jnp.max(h2, axis=1, keepdims=True)
  e = jnp.exp(h2 - m)
  lse = jnp.log(jnp.sum(e, axis=1, keepdims=True)) + m
  o[...] = h2 - lse


_tc3 = pl.pallas_call(
    _tc3_body,
    grid=(N // BN,),
    in_specs=[
        pl.BlockSpec((BN, D2), lambda i: (i, 0)),
        pl.BlockSpec((BN, D2), lambda i: (NPAD // BN + i, 0)),
        pl.BlockSpec((BN, D2), lambda i: (i, 0)),
        pl.BlockSpec((BN, 1), lambda i: (i, 0)),
        pl.BlockSpec((1, D2), lambda i: (0, 0)),
    ],
    out_specs=pl.BlockSpec((BN, D2), lambda i: (i, 0)),
    out_shape=jax.ShapeDtypeStruct((N, D2), jnp.float32),
)


def kernel(features, edges, W1, b1, W2, b2):
  src4 = edges[0].astype(jnp.int32).reshape(NW, NCHUNK, K)
  dst4 = edges[1].astype(jnp.int32).reshape(NW, NCHUNK, K)

  zerosd = jnp.zeros((NPAD, DW), jnp.float32)
  ones_k = jnp.ones((K, DW), jnp.float32)
  degp = _deg_kernel(dst4, zerosd, ones_k)

  hp, dis = _tc1(degp, degp, features, W1)

  zeros64 = jnp.zeros((NPAD, D1), jnp.bfloat16)
  p = _agg64(hp, src4, dst4, zeros64)
  g = _tc2(p, p, hp, dis, b1.reshape(1, D1), W2)

  zeros32 = jnp.zeros((NPAD, D2), jnp.bfloat16)
  q = _agg32(g, src4, dst4, zeros32)
  out = _tc3(q, q, g, dis, b2.reshape(1, D2))
  return out


# speedup vs baseline: 57.5252x; 1.0352x over previous
import functools

import jax
import jax.numpy as jnp
from jax import lax
from jax.experimental import pallas as pl
from jax.experimental.pallas import tpu as pltpu
from jax.experimental.pallas import tpu_sc as plsc

N = 10000
E = 320000
DF = 128
D1 = 64
D2 = 32
NPAD = 10000

NC = 2
NS = 16
NW = NC * NS
K = 125
NCHUNK = 80
RPT = NPAD // NS

_MESH = plsc.VectorSubcoreMesh(
    core_axis_name="c", subcore_axis_name="s", num_cores=NC, num_subcores=NS)
_SC_PARAMS = pltpu.CompilerParams(use_tc_tiling_on_sc=False)


DW = 8
NBUF = 8
NOUTER = NCHUNK // NBUF
DEGW = 6


def _make_deg_kernel():
  @functools.partial(
      pl.kernel,
      out_type=jax.ShapeDtypeStruct((NC * NPAD, 8), jnp.float32),
      mesh=_MESH,
      compiler_params=_SC_PARAMS,
      scratch_types=[
          pltpu.VMEM_SHARED((NPAD, DW), jnp.float32),
          pltpu.VMEM((NCHUNK, K), jnp.int32),
          pltpu.VMEM((K, DW), jnp.float32),
          pltpu.SemaphoreType.DMA,
      ],
  )
  def deg_kernel(dst4_hbm, zeros_hbm, ones_hbm, out_hbm, acc, didx3, ones_v,
                 dsem):
    c = lax.axis_index("c")
    s = lax.axis_index("s")
    r0 = s * RPT
    w = s * NC + c
    pltpu.sync_copy(zeros_hbm.at[pl.ds(r0, RPT), :], acc.at[pl.ds(r0, RPT), :])
    pltpu.sync_copy(dst4_hbm.at[w], didx3)
    pltpu.sync_copy(ones_hbm, ones_v)
    plsc.subcore_barrier()

    def drain_one():
      pltpu.make_async_copy(ones_v, acc.at[didx3.at[0]], dsem).wait()

    def loop(j, carry):
      pltpu.async_copy(ones_v, acc.at[didx3.at[j]], dsem, add=True)

      @pl.when(j >= DEGW)
      def _():
        drain_one()

      return carry

    lax.fori_loop(0, NCHUNK, loop, 0)

    def tail(i, carry):
      drain_one()
      return carry

    lax.fori_loop(0, DEGW, tail, 0)
    plsc.subcore_barrier()
    pltpu.sync_copy(acc.at[pl.ds(r0, RPT), pl.ds(0, 8)],
                    out_hbm.at[pl.ds(c * NPAD + r0, RPT), :])

  return deg_kernel


def _make_agg_kernel(d):
  scratch = [
      pltpu.VMEM_SHARED((NPAD, d), jnp.bfloat16),
      pltpu.VMEM((NCHUNK, K), jnp.int32),
      pltpu.VMEM((NCHUNK, K), jnp.int32),
  ]
  scratch += [pltpu.VMEM((K, d), jnp.bfloat16) for _ in range(NBUF)]
  scratch += [pltpu.SemaphoreType.DMA for _ in range(2 * NBUF)]

  @functools.partial(
      pl.kernel,
      out_type=jax.ShapeDtypeStruct((NC * NPAD, d), jnp.bfloat16),
      mesh=_MESH,
      compiler_params=_SC_PARAMS,
      scratch_types=scratch,
  )
  def agg_kernel(tab_hbm, src4_hbm, dst4_hbm, zeros_hbm, out_hbm,
                 acc, sidx3, didx3, *bufs):
    rows = bufs[:NBUF]
    gsem = bufs[NBUF:2 * NBUF]
    ssem = bufs[2 * NBUF:3 * NBUF]
    c = lax.axis_index("c")
    s = lax.axis_index("s")
    r0 = s * RPT
    w = s * NC + c
    pltpu.sync_copy(zeros_hbm.at[pl.ds(r0, RPT), :], acc.at[pl.ds(r0, RPT), :])
    pltpu.sync_copy(src4_hbm.at[w], sidx3)
    pltpu.sync_copy(dst4_hbm.at[w], didx3)
    plsc.subcore_barrier()

    def gfire(j, b):
      pltpu.async_copy(tab_hbm.at[sidx3.at[j]], rows[b], gsem[b])

    def gwait(b):
      pltpu.make_async_copy(
          tab_hbm.at[sidx3.at[0]], rows[b], gsem[b]).wait()

    def sfire(j, b):
      pltpu.async_copy(rows[b], acc.at[didx3.at[j]], ssem[b], add=True)

    def swait(b):
      pltpu.make_async_copy(rows[b], acc.at[didx3.at[0]], ssem[b]).wait()

    for b in range(NBUF - 1):
      gfire(b, b)

    def outer(t, carry):
      for b in range(NBUF):
        j = t * NBUF + b
        b1 = (b - 1) % NBUF
        gwait(b)
        sfire(j, b)

        @pl.when(j > 0)
        def _():
          swait(b1)

        @pl.when(j + NBUF - 1 < NCHUNK)
        def _():
          gfire(j + NBUF - 1, b1)

      return carry

    lax.fori_loop(0, NOUTER, outer, 0)
    swait((NCHUNK - 1) % NBUF)
    plsc.subcore_barrier()
    pltpu.sync_copy(acc.at[pl.ds(r0, RPT), :],
                    out_hbm.at[pl.ds(c * NPAD + r0, RPT), :])

  return agg_kernel


_deg_kernel = _make_deg_kernel()
_agg64 = _make_agg_kernel(D1)
_agg32 = _make_agg_kernel(D2)


BN = 2000


def _tc1_body(d0, d1, x, w1, hp_o, dis_o):
  deg = d0[...][:, :1] + d1[...][:, :1] + 1.0
  dis = lax.rsqrt(deg)
  h = jnp.dot(x[...], w1[...], preferred_element_type=jnp.float32)
  hp_o[...] = (h * dis).astype(jnp.bfloat16)
  dis_o[...] = dis


_tc1 = pl.pallas_call(
    _tc1_body,
    grid=(N // BN,),
    in_specs=[
        pl.BlockSpec((BN, 8), lambda i: (i, 0)),
        pl.BlockSpec((BN, 8), lambda i: (NPAD // BN + i, 0)),
        pl.BlockSpec((BN, DF), lambda i: (i, 0)),
        pl.BlockSpec((DF, D1), lambda i: (0, 0)),
    ],
    out_specs=[
        pl.BlockSpec((BN, D1), lambda i: (i, 0)),
        pl.BlockSpec((BN, 1), lambda i: (i, 0)),
    ],
    out_shape=[
        jax.ShapeDtypeStruct((N, D1), jnp.bfloat16),
        jax.ShapeDtypeStruct((N, 1), jnp.float32),
    ],
)


def _tc2_body(p0, p1, hp, dis, b1, w2, g_o):
  s = (p0[...].astype(jnp.float32) + p1[...].astype(jnp.float32)
       + hp[...].astype(jnp.float32))
  h1 = jnp.maximum(dis[...] * s + b1[...], 0.0)
  g = jnp.dot(h1, w2[...], preferred_element_type=jnp.float32) * dis[...]
  g_o[...] = g.astype(jnp.bfloat16)


_tc2 = pl.pallas_call(
    _tc2_body,
    grid=(N // BN,),
    in_specs=[
        pl.BlockSpec((BN, D1), lambda i: (i, 0)),
        pl.BlockSpec((BN, D1), lambda i: (NPAD // BN + i, 0)),
        pl.BlockSpec((BN, D1), lambda i: (i, 0)),
        pl.BlockSpec((BN, 1), lambda i: (i, 0)),
        pl.BlockSpec((1, D1), lambda i: (0, 0)),
        pl.BlockSpec((D1, D2), lambda i: (0, 0)),
    ],
    out_specs=pl.BlockSpec((BN, D2), lambda i: (i, 0)),
    out_shape=jax.ShapeDtypeStruct((N, D2), jnp.bfloat16),
)


def _tc3_body(q0, q1, g, dis, b2, o):
  s = (q0[...].astype(jnp.float32) + q1[...].astype(jnp.float32)
       + g[...].astype(jnp.float32))
  h2 = dis[...] * s + b2[...]
  m = jnp.max(h2, axis=1, keepdims=True)
  e = jnp.exp(h2 - m)
  lse = jnp.log(jnp.sum(e, axis=1, keepdims=True)) + m
  o[...] = h2 - lse


_tc3 = pl.pallas_call(
    _tc3_body,
    grid=(N // BN,),
    in_specs=[
        pl.BlockSpec((BN, D2), lambda i: (i, 0)),
        pl.BlockSpec((BN, D2), lambda i: (NPAD // BN + i, 0)),
        pl.BlockSpec((BN, D2), lambda i: (i, 0)),
        pl.BlockSpec((BN, 1), lambda i: (i, 0)),
        pl.BlockSpec((1, D2), lambda i: (0, 0)),
    ],
    out_specs=pl.BlockSpec((BN, D2), lambda i: (i, 0)),
    out_shape=jax.ShapeDtypeStruct((N, D2), jnp.float32),
)


def kernel(features, edges, W1, b1, W2, b2):
  src4 = edges[0].astype(jnp.int32).reshape(NW, NCHUNK, K)
  dst4 = edges[1].astype(jnp.int32).reshape(NW, NCHUNK, K)

  zerosd = jnp.zeros((NPAD, DW), jnp.float32)
  ones_k = jnp.ones((K, DW), jnp.float32)
  degp = _deg_kernel(dst4, zerosd, ones_k)

  hp, dis = _tc1(degp, degp, features, W1)

  zeros64 = jnp.zeros((NPAD, D1), jnp.bfloat16)
  p = _agg64(hp, src4, dst4, zeros64)
  g = _tc2(p, p, hp, dis, b1.reshape(1, D1), W2)

  zeros32 = jnp.zeros((NPAD, D2), jnp.bfloat16)
  q = _agg32(g, src4, dst4, zeros32)
  out = _tc3(q, q, g, dis, b2.reshape(1, D2))
  return out
